# baseline (speedup 1.0000x reference)
import functools
import jax
import jax.numpy as jnp
from jax.experimental import pallas as pl
from jax.experimental.pallas import tpu as pltpu

_BB = 64
_BD = 256


def _body(f_ref, m_ref, e_ref, o_ref):
    pe = jnp.maximum(e_ref[...], 0.0)
    o_ref[...] = f_ref[...] + pe[None, :, :] * m_ref[...][:, :, None]


def kernel(video_feats, video_masks, emb_table):
    B, L, D = video_feats.shape
    return pl.pallas_call(
        _body,
        grid=(B // _BB, D // _BD),
        in_specs=[
            pl.BlockSpec((_BB, L, _BD), lambda i, j: (i, 0, j)),
            pl.BlockSpec((_BB, L), lambda i, j: (i, 0)),
            pl.BlockSpec((L, _BD), lambda i, j: (0, j)),
        ],
        out_specs=pl.BlockSpec((_BB, L, _BD), lambda i, j: (i, 0, j)),
        out_shape=jax.ShapeDtypeStruct((B, L, D), video_feats.dtype),
        compiler_params=pltpu.CompilerParams(
            dimension_semantics=("parallel", "parallel"),
        ),
    )(video_feats, video_masks, emb_table)


# speedup vs baseline: 1.0557x; 1.0557x over previous
import functools

import jax
import jax.numpy as jnp
from jax.experimental import pallas as pl
from jax.experimental.pallas import tpu as pltpu

_BB = 56


def _body(f_ref, m_ref, e_ref, o_ref):
    pe = jnp.maximum(e_ref[...], 0.0)
    o_ref[...] = f_ref[...] + pe[None, :, :] * m_ref[...][:, :, None]


@functools.partial(jax.jit, donate_argnums=())
def kernel(video_feats, video_masks, emb_table):
    B, L, D = video_feats.shape
    grid = (pl.cdiv(B, _BB),)
    return pl.pallas_call(
        _body,
        grid=grid,
        in_specs=[
            pl.BlockSpec((_BB, L, D), lambda i: (i, 0, 0)),
            pl.BlockSpec((_BB, L), lambda i: (i, 0)),
            pl.BlockSpec((L, D), lambda i: (0, 0)),
        ],
        out_specs=pl.BlockSpec((_BB, L, D), lambda i: (i, 0, 0)),
        out_shape=jax.ShapeDtypeStruct((B, L, D), video_feats.dtype),
        compiler_params=pltpu.CompilerParams(
            dimension_semantics=("arbitrary",),
        ),
    )(video_feats, video_masks, emb_table)
